# Initial kernel scaffold; baseline (speedup 1.0000x reference)
import jax
import jax.numpy as jnp
from jax.experimental import pallas as pl


def kernel(input, batch_sample_xyz, sampling):
    raise NotImplementedError("write your pallas kernel here")


# speedup vs baseline: 8.5719x; 8.5719x over previous
import jax
import jax.numpy as jnp
from jax.experimental import pallas as pl
from jax.experimental.pallas import tpu as pltpu

_K = 32


def _topk_body(samp_ref, xyzt_ref, idx_ref):
    R = samp_ref.shape[1]
    N = xyzt_ref.shape[2]
    q = samp_ref[0]
    p = xyzt_ref[0]
    d = ((q[:, 0:1] - p[0:1, :]) ** 2
         + (q[:, 1:2] - p[1:2, :]) ** 2
         + (q[:, 2:3] - p[2:3, :]) ** 2)
    colidx = jax.lax.broadcasted_iota(jnp.int32, (R, N), 1)
    inf = jnp.float32(jnp.inf)
    cols = []
    for _ in range(_K):
        mv = jnp.min(d, axis=1, keepdims=True)
        cand = jnp.where(d == mv, colidx, jnp.int32(N))
        aidx = jnp.min(cand, axis=1, keepdims=True)
        cols.append(aidx)
        d = jnp.where(colidx == aidx, inf, d)
    idx_ref[0] = jnp.concatenate(cols, axis=1)


def _gather_body(idx_ref, x_ref, out_ref):
    S = idx_ref.shape[1]

    def qstep(i, carry):
        acc = x_ref[0, idx_ref[0, i, 0], :]
        for k in range(1, _K):
            acc = jnp.maximum(acc, x_ref[0, idx_ref[0, i, k], :])
        out_ref[0, i, :] = acc
        return carry

    jax.lax.fori_loop(0, S, qstep, 0)


def kernel(input, batch_sample_xyz, sampling):
    B, N, D = input.shape
    M = sampling.shape[1]
    xyzt = jnp.transpose(batch_sample_xyz, (0, 2, 1))

    R = min(128, M)
    idx = pl.pallas_call(
        _topk_body,
        grid=(B, M // R),
        in_specs=[
            pl.BlockSpec((1, R, 3), lambda b, i: (b, i, 0)),
            pl.BlockSpec((1, 3, N), lambda b, i: (b, 0, 0)),
        ],
        out_specs=pl.BlockSpec((1, R, _K), lambda b, i: (b, i, 0)),
        out_shape=jax.ShapeDtypeStruct((B, M, _K), jnp.int32),
    )(sampling, xyzt)

    S = min(256, M)
    out = pl.pallas_call(
        _gather_body,
        grid=(B, M // S),
        in_specs=[
            pl.BlockSpec((1, S, _K), lambda b, i: (b, i, 0),
                         memory_space=pltpu.SMEM),
            pl.BlockSpec((1, N, D), lambda b, i: (b, 0, 0)),
        ],
        out_specs=pl.BlockSpec((1, S, D), lambda b, i: (b, i, 0)),
        out_shape=jax.ShapeDtypeStruct((B, M, D), jnp.float32),
    )(idx, input)
    return out
